# baseline (speedup 1.0000x reference)
import functools
import math

import jax
import jax.numpy as jnp
from jax.experimental import pallas as pl
from jax.experimental.pallas import tpu as pltpu
from jax.experimental.pallas import tpu_sc as plsc

D_MODEL = 1024
NUM_EXPERTS = 8
HIDDEN = 4096
TOKENS = 2048
ASSIGN = 2 * TOKENS
EPAD = 128
BM = 128
NB = ASSIGN // BM + NUM_EXPERTS
NPAD = NB * BM

_INTERPRET = False


def _gate_kernel(x_ref, gw_ref, gb_ref, i0_ref, i1_ref, w0_ref, w1_ref,
                 ll_ref):
    x = x_ref[...]
    gw = gw_ref[...]
    logits = jnp.dot(x, gw, preferred_element_type=jnp.float32)
    logits = logits + gb_ref[...]
    lane = jax.lax.broadcasted_iota(jnp.int32, (TOKENS, EPAD), 1)
    valid = lane < NUM_EXPERTS
    logits = jnp.where(valid, logits, jnp.float32(-1e30))
    m = jnp.max(logits, axis=1, keepdims=True)
    p = jnp.where(valid, jnp.exp(logits - m), 0.0)
    s = jnp.sum(p, axis=1, keepdims=True)
    probs = p / s
    big = jnp.int32(EPAD)
    v0 = jnp.max(probs, axis=1, keepdims=True)
    i0 = jnp.min(jnp.where(probs == v0, lane, big), axis=1, keepdims=True)
    probs1 = jnp.where(lane == i0, jnp.float32(-1.0), probs)
    v1 = jnp.max(probs1, axis=1, keepdims=True)
    i1 = jnp.min(jnp.where(probs1 == v1, lane, big), axis=1, keepdims=True)
    denom = v0 + v1 + jnp.float32(1e-9)
    w0 = v0 / denom
    w1 = v1 / denom
    i0_ref[...] = jnp.broadcast_to(i0, (TOKENS, EPAD))
    i1_ref[...] = jnp.broadcast_to(i1, (TOKENS, EPAD))
    w0_ref[...] = jnp.broadcast_to(w0, (TOKENS, EPAD))
    w1_ref[...] = jnp.broadcast_to(w1, (TOKENS, EPAD))
    pm = jnp.sum(probs, axis=0, keepdims=True) / jnp.float32(TOKENS)
    diff = jnp.where(valid[0:1, :], pm - jnp.float32(1.0 / NUM_EXPERTS), 0.0)
    ll = jnp.sum(diff * diff) / jnp.float32(NUM_EXPERTS)
    ll_ref[...] = jnp.full((1, EPAD), ll, dtype=jnp.float32)


def _gate(x2d, gate_W, gate_b):
    gwp = jnp.zeros((D_MODEL, EPAD), jnp.float32).at[:, :NUM_EXPERTS].set(
        gate_W)
    gbp = jnp.zeros((1, EPAD), jnp.float32).at[0, :NUM_EXPERTS].set(gate_b)
    outs = pl.pallas_call(
        _gate_kernel,
        out_shape=(
            jax.ShapeDtypeStruct((TOKENS, EPAD), jnp.int32),
            jax.ShapeDtypeStruct((TOKENS, EPAD), jnp.int32),
            jax.ShapeDtypeStruct((TOKENS, EPAD), jnp.float32),
            jax.ShapeDtypeStruct((TOKENS, EPAD), jnp.float32),
            jax.ShapeDtypeStruct((1, EPAD), jnp.float32),
        ),
        interpret=_INTERPRET,
    )(x2d, gwp, gbp)
    return outs


NH = 2
BH = HIDDEN // NH


def _ffn_kernel(be_ref, xs_ref, w1_ref, b1_ref, w2_ref, b2_ref, out_ref,
                w1bf_ref, w2bf_ref, acc_ref):
    h = pl.program_id(0)
    b = pl.program_id(1)
    prev = be_ref[jnp.maximum(b - 1, 0)]
    changed = jnp.logical_or(b == 0, be_ref[b] != prev)

    @pl.when(changed)
    def _cast():
        w1bf_ref[...] = w1_ref[0].astype(jnp.bfloat16)
        w2bf_ref[...] = w2_ref[0].astype(jnp.bfloat16)

    xb = xs_ref[...]
    hm = jnp.dot(xb, w1bf_ref[...], preferred_element_type=jnp.float32)
    hm = hm + b1_ref[0]
    hm = 0.5 * hm * (1.0 + jax.lax.erf(hm * jnp.float32(1.0 / math.sqrt(2.0))))
    part = jnp.dot(hm.astype(jnp.bfloat16), w2bf_ref[...],
                   preferred_element_type=jnp.float32)

    @pl.when(h == 0)
    def _store():
        acc_ref[pl.ds(b * BM, BM), :] = (part + b2_ref[0]).astype(
            jnp.bfloat16)

    @pl.when(h == NH - 1)
    def _final():
        out_ref[...] = part + acc_ref[pl.ds(b * BM, BM), :].astype(
            jnp.float32)


def _grouped_ffn(block_expert, xs, W1, b1, W2, b2):
    grid_spec = pltpu.PrefetchScalarGridSpec(
        num_scalar_prefetch=1,
        grid=(NH, NB),
        in_specs=[
            pl.BlockSpec((BM, D_MODEL), lambda h, b, be: (b, 0)),
            pl.BlockSpec((1, D_MODEL, BH), lambda h, b, be: (be[b], 0, h)),
            pl.BlockSpec((1, 1, BH), lambda h, b, be: (be[b], 0, h)),
            pl.BlockSpec((1, BH, D_MODEL), lambda h, b, be: (be[b], h, 0)),
            pl.BlockSpec((1, 1, D_MODEL), lambda h, b, be: (be[b], 0, 0)),
        ],
        out_specs=pl.BlockSpec((BM, D_MODEL), lambda h, b, be: (b, 0)),
        scratch_shapes=[
            pltpu.VMEM((D_MODEL, BH), jnp.bfloat16),
            pltpu.VMEM((BH, D_MODEL), jnp.bfloat16),
            pltpu.VMEM((NPAD, D_MODEL), jnp.bfloat16),
        ],
    )
    return pl.pallas_call(
        _ffn_kernel,
        grid_spec=grid_spec,
        out_shape=jax.ShapeDtypeStruct((NPAD, D_MODEL), jnp.float32),
        compiler_params=pltpu.CompilerParams(
            dimension_semantics=("arbitrary", "arbitrary")),
        interpret=_INTERPRET,
    )(block_expert, xs, W1,
      b1.reshape(NUM_EXPERTS, 1, HIDDEN), W2,
      b2.reshape(NUM_EXPERTS, 1, D_MODEL))


NC = 2
NS = 16
ROWS_W = NPAD // (NC * NS)
POS_W = ASSIGN // (NC * NS)
GCH = 2
ROWS_CH = ROWS_W // GCH
XW = D_MODEL // 2
NBPAD = 48


_GDN = jax.lax.GatherDimensionNumbers(
    offset_dims=(), collapsed_slice_dims=(0,), start_index_map=(0,))


def _splat16(vec, idx16):
    return jax.lax.gather(
        vec, idx16[:, None], dimension_numbers=_GDN, slice_sizes=(1,),
        mode=jax.lax.GatherScatterMode.PROMISE_IN_BOUNDS)


def _prefix16(x, iota):
    y = x
    for k in (1, 2, 4, 8):
        sh = _splat16(y, jnp.maximum(iota - k, 0))
        y = y + jnp.where(iota >= k, sh, 0)
    return y


def _sc_offsets(counts, iota):
    padded = jnp.bitwise_and(counts + (BM - 1), jnp.int32(~(BM - 1)))
    return _prefix16(padded, iota) - padded


SEG = ASSIGN // (NC * NS)
SEGCH = SEG // 16


def _sc_route_kernel(e_hbm, x_hbm, xs_hbm, pos_hbm, be_hbm,
                     e_v, cnt32_v, ccnt_v, runv_v, pos_out_v, be_v,
                     xrow_v, idx16_v, scal_s, sh_cnt32, sem):
    scal_s[2] = jax.lax.axis_index("c")
    scal_s[3] = jax.lax.axis_index("s")
    cid = scal_s[2]
    sid = scal_s[3]
    w = cid * NS + sid
    iota = jax.lax.broadcasted_iota(jnp.int32, (16,), 0)
    ones = jnp.ones((16,), jnp.int32)
    zeros = jnp.zeros((16,), jnp.int32)
    segoff = pl.multiple_of(w * SEG, SEG)

    sbase = pl.multiple_of(sid * SEG, SEG)
    pltpu.sync_copy(e_hbm.at[pl.ds(sbase, SEG)], e_v.at[pl.ds(0, SEG)])
    pltpu.sync_copy(e_hbm.at[pl.ds(sbase + NS * SEG, SEG)],
                    e_v.at[pl.ds(SEG, SEG)])
    for half in range(2):
        ccnt = zeros
        for c in range(SEGCH):
            ev = e_v[pl.ds(half * SEG + c * 16, 16)]
            for e in range(NUM_EXPERTS):
                inc = _splat16(
                    _prefix16(jnp.where(ev == e, ones, zeros), iota),
                    jnp.full((16,), 15, jnp.int32))
                ccnt = jnp.where(iota == e, ccnt + inc, ccnt)
        ccnt_v[...] = ccnt
        pltpu.sync_copy(ccnt_v, sh_cnt32.at[sid + half * NS])
    plsc.subcore_barrier()

    pltpu.sync_copy(sh_cnt32, cnt32_v)
    total = zeros
    segbase = zeros
    wv = jnp.full((16,), w, dtype=jnp.int32)
    for s in range(NC * NS):
        row = cnt32_v[s, :]
        mask = jnp.where(jnp.full((16,), s, jnp.int32) < wv, ones, zeros)
        segbase = segbase + row * mask
        total = total + row
    offsets = _sc_offsets(total, iota)
    segbase = segbase + offsets

    @pl.when(cid + jnp.abs(sid - NUM_EXPERTS) == 0)
    def _bmap():
        iota_ = jax.lax.broadcasted_iota(jnp.int32, (16,), 0)
        tot2 = jnp.zeros((16,), jnp.int32)
        for s in range(NC * NS):
            tot2 = tot2 + cnt32_v[s, :]
        offs = _sc_offsets(tot2, iota_)
        for c in range(NBPAD // 16):
            bv = (c * 16 + iota_) * BM
            cnt = jnp.zeros((16,), jnp.int32)
            for ei in range(NUM_EXPERTS):
                obv = _splat16(offs, jnp.full((16,), ei, jnp.int32))
                cnt = cnt + jnp.where(bv >= obv,
                                      jnp.ones((16,), jnp.int32),
                                      jnp.zeros((16,), jnp.int32))
            be_v[pl.ds(c * 16, 16)] = cnt - 1
        pltpu.sync_copy(be_v, be_hbm)

    runv_v[...] = segbase
    for c in range(SEGCH):
        ev = e_v[pl.ds(cid * SEG + c * 16, 16)]
        basel = _splat16(runv_v[...], ev)
        rank = zeros
        for k in range(1, 16):
            sh = _splat16(ev, jnp.maximum(iota - k, 0))
            a = jnp.where(iota >= k, ones, zeros)
            b = jnp.where(sh == ev, ones, zeros)
            rank = rank + a * b
        posv = basel + rank
        pos_out_v[pl.ds(c * 16, 16)] = posv
        tokv = jax.lax.div(jnp.full((16,), segoff + c * 16, jnp.int32)
                           + iota, jnp.full((16,), 2, jnp.int32))
        idx16_v[...] = tokv
        pltpu.async_copy(x_hbm.at[idx16_v], xrow_v, sem).wait()
        pltpu.sync_copy(xrow_v, xs_hbm.at[posv])
        run = runv_v[...]
        for e in range(NUM_EXPERTS):
            inc = _splat16(
                _prefix16(jnp.where(ev == e, ones, zeros), iota),
                jnp.full((16,), 15, jnp.int32))
            run = jnp.where(iota == e, run + inc, run)
        runv_v[...] = run
    pltpu.sync_copy(pos_out_v, pos_hbm.at[pl.ds(segoff, SEG)])


def _sc_route_gather(e_flat, x_i32):
    mesh = plsc.VectorSubcoreMesh(core_axis_name="c", subcore_axis_name="s")
    f = functools.partial(
        pl.kernel,
        mesh=mesh,
        out_type=(
            jax.ShapeDtypeStruct((NPAD, XW), jnp.int32),
            jax.ShapeDtypeStruct((ASSIGN,), jnp.int32),
            jax.ShapeDtypeStruct((NBPAD,), jnp.int32),
        ),
        scratch_types=[
            pltpu.VMEM((2 * SEG,), jnp.int32),
            pltpu.VMEM((NC * NS, 16), jnp.int32),
            pltpu.VMEM((16,), jnp.int32),
            pltpu.VMEM((16,), jnp.int32),
            pltpu.VMEM((SEG,), jnp.int32),
            pltpu.VMEM((NBPAD,), jnp.int32),
            pltpu.VMEM((16, XW), jnp.int32),
            pltpu.VMEM((16,), jnp.int32),
            pltpu.SMEM((4,), jnp.int32),
            pltpu.VMEM_SHARED((NC * NS, 16), jnp.int32),
            pltpu.SemaphoreType.DMA,
        ],
        )(_sc_route_kernel)
    return f(e_flat, x_i32)


def _route(e_flat):
    oh = (e_flat[:, None] == jnp.arange(NUM_EXPERTS)[None, :]).astype(
        jnp.int32)
    counts = oh.sum(axis=0)
    padded = ((counts + BM - 1) // BM) * BM
    offsets = jnp.concatenate(
        [jnp.zeros((1,), jnp.int32), jnp.cumsum(padded)[:-1]]).astype(
            jnp.int32)
    rank = jnp.cumsum(oh, axis=0) - oh
    rank_a = jnp.take_along_axis(rank, e_flat[:, None], axis=1)[:, 0]
    pos = (offsets[e_flat] + rank_a).astype(jnp.int32)
    tok = jnp.arange(ASSIGN, dtype=jnp.int32) // 2
    row_token = jnp.zeros((NPAD,), jnp.int32).at[pos].set(tok)
    off_blk = offsets // BM
    block_expert = (jnp.arange(NB, dtype=jnp.int32)[:, None]
                    >= off_blk[None, :]).astype(jnp.int32).sum(axis=1) - 1
    return pos, row_token, block_expert.astype(jnp.int32)


def kernel(x, gate_W, gate_b, W1, b1, W2, b2):
    bsz, seq_len, d_model = x.shape
    x2d = x.reshape(TOKENS, D_MODEL)
    i0b, i1b, w0b, w1b, llb = _gate(x2d, gate_W, gate_b)
    i0 = i0b[:, 0]
    i1 = i1b[:, 0]
    e_flat = jnp.stack([i0, i1], axis=1).reshape(ASSIGN)
    w_flat = jnp.stack([w0b[:, 0], w1b[:, 0]], axis=1).reshape(ASSIGN)

    x_bf = x2d.astype(jnp.bfloat16)
    x_i32 = jax.lax.bitcast_convert_type(
        x_bf.reshape(TOKENS, XW, 2), jnp.int32)
    xs_i32, pos, be48 = _sc_route_gather(e_flat, x_i32)
    pos, _row_token, block_expert = _route(e_flat)
    del be48
    xs = jax.lax.bitcast_convert_type(
        xs_i32, jnp.bfloat16).reshape(NPAD, D_MODEL)
    rows = _grouped_ffn(block_expert, xs, W1, b1, W2, b2)
    gathered = rows[pos].reshape(TOKENS, 2, D_MODEL)
    moe = (gathered * w_flat.reshape(TOKENS, 2, 1)).sum(axis=1)
    moe_out = moe.reshape(bsz, seq_len, d_model)
    load_loss = llb[0, 0].reshape(())
    return moe_out, load_loss
